# baseline (speedup 1.0000x reference)
import jax
import jax.numpy as jnp
from jax import lax
from jax.experimental import pallas as pl
from jax.experimental.pallas import tpu as pltpu
from jax.experimental.pallas import tpu_sc as plsc

EMBED_DIM = 128
CHUNK = 128
NBUF = 5


def kernel(inputs, table):
    batch, seq_len, _ = inputs.shape
    num_idx = batch * seq_len
    num_chunks = num_idx // CHUNK

    mesh = plsc.VectorSubcoreMesh(core_axis_name="core",
                                  subcore_axis_name="subcore")
    num_workers = mesh.num_cores * mesh.num_subcores
    nch = num_chunks // num_workers
    idx3d = inputs.reshape(num_workers, nch, CHUNK)

    @pl.kernel(
        out_type=jax.ShapeDtypeStruct((num_idx, EMBED_DIM), table.dtype),
        mesh=mesh,
        scratch_types=[
            pltpu.VMEM((nch, CHUNK), jnp.int32),
            pltpu.VMEM((NBUF, CHUNK, EMBED_DIM), jnp.float32),
            pltpu.VMEM_SHARED((mesh.num_subcores, 2, CHUNK, EMBED_DIM),
                              jnp.float32),
            pltpu.SemaphoreType.DMA((NBUF,)),
            pltpu.SemaphoreType.DMA((NBUF,)),
            pltpu.SemaphoreType.DMA((NBUF,)),
        ],
    )
    def gather_kernel(table_hbm, i_hbm, o_hbm, idx_v, buf_v, sbuf,
                      gsem, esem, wsem):
        sid = lax.axis_index("subcore")
        wid = sid * mesh.num_cores + lax.axis_index("core")
        c0 = wid * nch

        pltpu.sync_copy(i_hbm.at[wid], idx_v)

        def gather_start(cl, b):
            pltpu.async_copy(table_hbm.at[idx_v.at[cl]], buf_v.at[b],
                             gsem.at[b])

        def gather_wait(b):
            pltpu.make_async_copy(table_hbm.at[idx_v.at[0]], buf_v.at[b],
                                  gsem.at[b]).wait()

        def evac(b):
            pltpu.async_copy(buf_v.at[b], sbuf.at[sid, b % 2], esem.at[b])
            pltpu.make_async_copy(buf_v.at[b], sbuf.at[sid, b % 2],
                                  esem.at[b]).wait()

        def write_start(cl, b):
            pltpu.async_copy(sbuf.at[sid, b % 2],
                             o_hbm.at[pl.ds((c0 + cl) * CHUNK, CHUNK)],
                             wsem.at[b])

        def write_wait(b):
            pltpu.make_async_copy(sbuf.at[sid, b % 2],
                                  o_hbm.at[pl.ds(c0 * CHUNK, CHUNK)],
                                  wsem.at[b]).wait()

        for b in range(NBUF):
            gather_start(b, b)

        @pl.loop(0, nch - NBUF, step=NBUF)
        def _(j):
            for b in range(NBUF):
                gather_wait(b)
                evac(b)
                gather_start(j + b + NBUF, b)
                write_start(j + b, b)
                write_wait(b)

        for b in range(NBUF):
            gather_wait(b)
            evac(b)
            write_start(nch - NBUF + b, b)
            write_wait(b)

    out = gather_kernel(table, idx3d)
    return out.reshape(batch, seq_len, EMBED_DIM)


# speedup vs baseline: 1.0272x; 1.0272x over previous
import jax
import jax.numpy as jnp
from jax import lax
from jax.experimental import pallas as pl
from jax.experimental.pallas import tpu as pltpu
from jax.experimental.pallas import tpu_sc as plsc

EMBED_DIM = 128
CHUNK = 128
NBUF = 5


def kernel(inputs, table):
    batch, seq_len, _ = inputs.shape
    num_idx = batch * seq_len
    num_chunks = num_idx // CHUNK

    mesh = plsc.VectorSubcoreMesh(core_axis_name="core",
                                  subcore_axis_name="subcore")
    num_workers = mesh.num_cores * mesh.num_subcores
    nch = num_chunks // num_workers
    idx3d = inputs.reshape(num_workers, nch, CHUNK)

    @pl.kernel(
        out_type=jax.ShapeDtypeStruct((num_idx, EMBED_DIM), table.dtype),
        mesh=mesh,
        scratch_types=[
            pltpu.VMEM((nch, CHUNK), jnp.int32),
            pltpu.VMEM((NBUF, CHUNK, EMBED_DIM), jnp.float32),
            pltpu.SemaphoreType.DMA((NBUF,)),
            pltpu.SemaphoreType.DMA((NBUF,)),
        ],
    )
    def gather_kernel(table_hbm, i_hbm, o_hbm, idx_v, buf_v, gsem, wsem):
        wid = lax.axis_index("subcore") * mesh.num_cores + lax.axis_index("core")
        c0 = wid * nch

        pltpu.sync_copy(i_hbm.at[wid], idx_v)

        def gather_start(cl, b):
            pltpu.async_copy(table_hbm.at[idx_v.at[cl]], buf_v.at[b],
                             gsem.at[b])

        def gather_wait(b):
            pltpu.make_async_copy(table_hbm.at[idx_v.at[0]], buf_v.at[b],
                                  gsem.at[b]).wait()

        def write_start(cl, b):
            pltpu.async_copy(buf_v.at[b],
                             o_hbm.at[pl.ds((c0 + cl) * CHUNK, CHUNK)],
                             wsem.at[b])

        def write_wait(b):
            pltpu.make_async_copy(buf_v.at[b],
                                  o_hbm.at[pl.ds(c0 * CHUNK, CHUNK)],
                                  wsem.at[b]).wait()

        for b in range(NBUF):
            gather_start(b, b)

        @pl.loop(0, nch - NBUF, step=NBUF)
        def _(j):
            for b in range(NBUF):
                gather_wait(b)
                write_start(j + b, b)
                write_wait(b)
                gather_start(j + b + NBUF, b)

        for b in range(NBUF):
            gather_wait(b)
            write_start(nch - NBUF + b, b)
            write_wait(b)

    out = gather_kernel(table, idx3d)
    return out.reshape(batch, seq_len, EMBED_DIM)
